# baseline (speedup 1.0000x reference)
import functools

import jax
import jax.numpy as jnp
from jax import lax
from jax.experimental import pallas as pl
from jax.experimental.pallas import tpu as pltpu
from jax.experimental.pallas import tpu_sc as plsc

H, L, P, NQ, D = 8, 4, 4, 1024, 256
SLEV = (128.0, 64.0, 32.0, 16.0)
START = (0, 16384, 20480, 21504)
NROWS = 32 * 4096
SC_CH = 128
SC_NCH = 32


def _idx_body(q2_ref, rp_ref, w_ref, b_ref, out_ref):
    ot = lax.dot_general(w_ref[...], q2_ref[...], (((1,), (1,)), ((), ())),
                         preferred_element_type=jnp.float32) + b_ref[...]
    ot3 = ot.reshape(128, 2, NQ)
    for g in range(32):
        lvl = g % 4
        s = SLEV[lvl]
        xg = ot3[4 * g:4 * g + 4, 0, :]
        yg = ot3[4 * g:4 * g + 4, 1, :]
        lx = jnp.clip(rp_ref[lvl, 0:1, :] + xg * (1.0 / s), 0.0, 0.999)
        ly = jnp.clip(rp_ref[lvl, 1:2, :] + yg * (1.0 / s), 0.0, 0.999)
        ix = (lx * s).astype(jnp.int32)
        iy = (ly * s).astype(jnp.int32)
        out_ref[g] = ix + iy * int(s) + START[lvl]


@functools.lru_cache(maxsize=None)
def _make_sc_gather(nrows):
    ch = 64
    nbuf = 4
    nch = nrows // (32 * ch)
    rpw = nch * ch
    nit = nch // nbuf

    def body(table_hbm, idx_hbm, out_hbm, idx_v, b0, b1, b2, b3,
             si0, si1, si2, si3, so0, so1, so2, so3):
        bufs = (b0, b1, b2, b3)
        sin = (si0, si1, si2, si3)
        sout = (so0, so1, so2, so3)
        w = lax.axis_index("s") * 2 + lax.axis_index("c")
        pltpu.sync_copy(idx_hbm.at[w], idx_v)
        base = w * rpw
        for b in range(nbuf):
            pltpu.async_copy(table_hbm.at[idx_v.at[b]], bufs[b], sin[b])

        def quad(t, carry):
            cb = nbuf * t
            for b in range(nbuf):
                pltpu.make_async_copy(
                    table_hbm.at[idx_v.at[cb + b]], bufs[b], sin[b]).wait()
                pltpu.async_copy(
                    bufs[b], out_hbm.at[pl.ds(base + (cb + b) * ch, ch)],
                    sout[b])

            @pl.when(t < nit - 1)
            def _():
                for b in range(nbuf):
                    pltpu.make_async_copy(
                        bufs[b], out_hbm.at[pl.ds(base + (cb + b) * ch, ch)],
                        sout[b]).wait()
                    pltpu.async_copy(
                        table_hbm.at[idx_v.at[cb + b + nbuf]], bufs[b], sin[b])

            return carry

        lax.fori_loop(0, nit, quad, 0)
        for b in range(nbuf):
            c = nch - nbuf + b
            pltpu.make_async_copy(
                bufs[b], out_hbm.at[pl.ds(base + c * ch, ch)], sout[b]).wait()

    return functools.partial(
        pl.kernel,
        mesh=plsc.VectorSubcoreMesh(core_axis_name="c", subcore_axis_name="s"),
        out_type=jax.ShapeDtypeStruct((nrows, D // 2), jnp.int32),
        scratch_types=[pltpu.VMEM((nch, ch), jnp.int32)]
        + [pltpu.VMEM((ch, D // 2), jnp.int32)] * nbuf
        + [pltpu.SemaphoreType.DMA] * (2 * nbuf),
    )(body)


def _head_body(hoff, g_ref, acc_ref, q2_ref, qp_ref, aw_ref, awx_ref,
               vw0_ref, vw1_ref, vb0_ref, vb1_ref, ak_ref, hm_ref, out_ref):
    h = pl.program_id(0)
    q2 = q2_ref[...]
    lane_iota = lax.broadcasted_iota(jnp.int32, (NQ, 256), 1)
    q16 = lax.broadcasted_iota(jnp.int32, (NQ, 256), 0) // 16
    slot_cols = []
    for lvl in range(L):
        simil = aw_ref[lvl]
        drs = []
        for r in range(16):
            kr = g_ref[lvl, 256 * r:256 * (r + 1), :]
            dr = lax.dot_general(qp_ref[r], kr, (((1,), (0,)), ((), ())),
                                 preferred_element_type=jnp.float32)
            drs.append(dr.reshape(64, 1, 256))
        d2 = jnp.concatenate(drs, axis=1).reshape(NQ, 256)
        t = lax.dot_general(d2, simil, (((1,), (1,)), ((), ())),
                            preferred_element_type=jnp.float32)
        cols = [jnp.sum(jnp.where(lane_iota == (q16 + p * 64), t, 0.0),
                        axis=1, keepdims=True) for p in range(4)]
        slot_cols.append(jnp.concatenate(cols, axis=1))
    ak = ak_ref[...]
    ki2 = lax.dot_general(ak, awx_ref[0], (((1,), (1,)), ((), ())),
                          preferred_element_type=jnp.float32)
    attn_add = lax.dot_general(q2, ki2, (((1,), (1,)), ((), ())),
                               preferred_element_type=jnp.float32)
    logits = jnp.concatenate(slot_cols + [attn_add], axis=1)
    m = jnp.max(logits, axis=1, keepdims=True)
    e = jnp.exp(logits - m)
    a = e / jnp.sum(e, axis=1, keepdims=True)
    wsum = jnp.zeros((NQ, 256), jnp.float32)
    for lvl in range(L):
        for p in range(4):
            wsum = wsum + a[:, lvl * 4 + p:lvl * 4 + p + 1] * \
                g_ref[lvl, 1024 * p:1024 * (p + 1), :].astype(jnp.float32)
    a16 = jnp.sum(a[:, :16], axis=1, keepdims=True)
    v_main = lax.dot_general(wsum, vw0_ref[0], (((1,), (1,)), ((), ())),
                             preferred_element_type=jnp.float32) + a16 * vb0_ref[0]
    v2 = lax.dot_general(ak, vw1_ref[0], (((1,), (1,)), ((), ())),
                         preferred_element_type=jnp.float32) + vb1_ref[0]
    v_add = lax.dot_general(a[:, 16:], v2, (((1,), (0,)), ((), ())),
                            preferred_element_type=jnp.float32)
    hm = hm_ref[...]
    hme = jnp.exp(hm - jnp.max(hm, axis=0, keepdims=True))
    hw = hme / jnp.sum(hme, axis=0, keepdims=True)
    hsel = lax.broadcasted_iota(jnp.int32, (9, 1), 0) == h + hoff
    hwh = jnp.sum(jnp.where(hsel, hw, 0.0), axis=0, keepdims=True)
    contrib = (v_main + v_add) * hwh

    @pl.when(h == 0)
    def _():
        base = acc_ref[...] + contrib
        if hoff == 0:
            base = base + q2 * hw[8:9, :]
        out_ref[...] = base

    @pl.when(h != 0)
    def _():
        out_ref[...] = out_ref[...] + contrib


def _compute_idx(q2, rp_t, w, b2d):
    return pl.pallas_call(
        _idx_body,
        out_shape=jax.ShapeDtypeStruct((32, 4, NQ), jnp.int32),
    )(q2, rp_t, w, b2d)


def _head_stage(hoff, nh, g, acc, q2, qp, attn_w, value_w, vb3, ak, hm_t):
    return pl.pallas_call(
        functools.partial(_head_body, hoff),
        grid=(nh,),
        in_specs=[
            pl.BlockSpec((4, 4096, 256), lambda h: (h, 0, 0)),
            pl.BlockSpec((NQ, 256), lambda h: (0, 0)),
            pl.BlockSpec((NQ, 256), lambda h: (0, 0)),
            pl.BlockSpec((16, 64, 256), lambda h: (0, 0, 0)),
            pl.BlockSpec((4, 256, 256), lambda h: (h + hoff, 0, 0)),
            pl.BlockSpec((1, 256, 256), lambda h: (4 * (h + hoff) + 4, 0, 0)),
            pl.BlockSpec((1, 256, 256), lambda h: (2 * (h + hoff), 0, 0)),
            pl.BlockSpec((1, 256, 256), lambda h: (2 * (h + hoff) + 1, 0, 0)),
            pl.BlockSpec((1, 1, 256), lambda h: (2 * (h + hoff), 0, 0)),
            pl.BlockSpec((1, 1, 256), lambda h: (2 * (h + hoff) + 1, 0, 0)),
            pl.BlockSpec((64, 256), lambda h: (0, 0)),
            pl.BlockSpec((9, 256), lambda h: (0, 0)),
        ],
        out_specs=pl.BlockSpec((NQ, 256), lambda h: (0, 0)),
        out_shape=jax.ShapeDtypeStruct((NQ, 256), jnp.float32),
    )(g, acc, q2, qp, attn_w, attn_w, value_w, value_w, vb3, vb3, ak, hm_t)


def kernel(query, reference_points, input_flatten, input_spatial_shapes,
           add_keys, input_level_start_index, sampling_offsets_W,
           sampling_offsets_b, attn_W, value_W, value_b, head_mixer_W):
    q2 = query[0]
    rp_t = reference_points[0].transpose(1, 2, 0)
    b2d = sampling_offsets_b.reshape(256, 1)
    idx = _compute_idx(q2, rp_t, sampling_offsets_W, b2d)
    idxf = idx.reshape(NROWS)
    half = NROWS // 2
    gather = _make_sc_gather(half)
    table = lax.bitcast_convert_type(
        input_flatten[0].astype(jnp.bfloat16).reshape(21760, 128, 2),
        jnp.int32)
    g1 = gather(table, idxf[:half].reshape(32, 32, 64))
    g2 = gather(table, idxf[half:].reshape(32, 32, 64))
    g1 = lax.bitcast_convert_type(g1, jnp.bfloat16).reshape(half, 256)
    g2 = lax.bitcast_convert_type(g2, jnp.bfloat16).reshape(half, 256)
    qp = q2.reshape(64, 16, 256).transpose(1, 0, 2).astype(jnp.bfloat16)
    vb3 = value_b.reshape(16, 1, 256)
    hm_t = head_mixer_W.T
    ak = add_keys[0]
    zero = jnp.zeros((NQ, 256), jnp.float32)
    o1 = _head_stage(0, 4, g1.reshape(16, 4096, 256), zero, q2, qp,
                     attn_W, value_W, vb3, ak, hm_t)
    out = _head_stage(4, 4, g2.reshape(16, 4096, 256), o1, q2, qp,
                      attn_W, value_W, vb3, ak, hm_t)
    return out[None]


# speedup vs baseline: 1.3297x; 1.3297x over previous
import functools

import jax
import jax.numpy as jnp
from jax import lax
from jax.experimental import pallas as pl
from jax.experimental.pallas import tpu as pltpu
from jax.experimental.pallas import tpu_sc as plsc

H, L, P, NQ, D = 8, 4, 4, 1024, 256
SLEV = (128.0, 64.0, 32.0, 16.0)
START = (0, 16384, 20480, 21504)
NROWS = 32 * 4096
SC_CH = 128
SC_NCH = 32


def _idx_body(q2_ref, rp_ref, w_ref, b_ref, out_ref):
    ot = lax.dot_general(w_ref[...], q2_ref[...], (((1,), (1,)), ((), ())),
                         preferred_element_type=jnp.float32) + b_ref[...]
    ot3 = ot.reshape(128, 2, NQ)
    for g in range(32):
        lvl = g % 4
        s = SLEV[lvl]
        xg = ot3[4 * g:4 * g + 4, 0, :]
        yg = ot3[4 * g:4 * g + 4, 1, :]
        lx = jnp.clip(rp_ref[lvl, 0:1, :] + xg * (1.0 / s), 0.0, 0.999)
        ly = jnp.clip(rp_ref[lvl, 1:2, :] + yg * (1.0 / s), 0.0, 0.999)
        ix = (lx * s).astype(jnp.int32)
        iy = (ly * s).astype(jnp.int32)
        out_ref[g] = ix + iy * int(s) + START[lvl]


@functools.lru_cache(maxsize=None)
def _make_sc_gather(nrows):
    ch = 64
    nbuf = 4
    nch = nrows // (32 * ch)
    rpw = nch * ch
    nit = nch // nbuf

    def body(table_hbm, idx_hbm, out_hbm, idx_v, b0, b1, b2, b3,
             si0, si1, si2, si3, so0, so1, so2, so3):
        bufs = (b0, b1, b2, b3)
        sin = (si0, si1, si2, si3)
        sout = (so0, so1, so2, so3)
        w = lax.axis_index("s") * 2 + lax.axis_index("c")
        pltpu.sync_copy(idx_hbm.at[w], idx_v)
        base = w * rpw
        for b in range(nbuf):
            pltpu.async_copy(table_hbm.at[idx_v.at[b]], bufs[b], sin[b])

        def quad(t, carry):
            cb = nbuf * t
            for b in range(nbuf):
                pltpu.make_async_copy(
                    table_hbm.at[idx_v.at[cb + b]], bufs[b], sin[b]).wait()
                pltpu.async_copy(
                    bufs[b], out_hbm.at[pl.ds(base + (cb + b) * ch, ch)],
                    sout[b])

            @pl.when(t < nit - 1)
            def _():
                for b in range(nbuf):
                    pltpu.make_async_copy(
                        bufs[b], out_hbm.at[pl.ds(base + (cb + b) * ch, ch)],
                        sout[b]).wait()
                    pltpu.async_copy(
                        table_hbm.at[idx_v.at[cb + b + nbuf]], bufs[b], sin[b])

            return carry

        lax.fori_loop(0, nit, quad, 0)
        for b in range(nbuf):
            c = nch - nbuf + b
            pltpu.make_async_copy(
                bufs[b], out_hbm.at[pl.ds(base + c * ch, ch)], sout[b]).wait()

    return functools.partial(
        pl.kernel,
        mesh=plsc.VectorSubcoreMesh(core_axis_name="c", subcore_axis_name="s"),
        out_type=jax.ShapeDtypeStruct((nrows, D // 2), jnp.int32),
        scratch_types=[pltpu.VMEM((nch, ch), jnp.int32)]
        + [pltpu.VMEM((ch, D // 2), jnp.int32)] * nbuf
        + [pltpu.SemaphoreType.DMA] * (2 * nbuf),
    )(body)


def _head_body(hoff, g_ref, acc_ref, q2_ref, qp_ref, awe_ref, awo_ref,
               awx_ref, vwe_ref, vwo_ref, vw1_ref, vb0_ref, vb1_ref,
               ak_ref, hm_ref, out_ref):
    h = pl.program_id(0)
    q2 = q2_ref[...]
    lane_iota = lax.broadcasted_iota(jnp.int32, (NQ, 256), 1)
    q16 = lax.broadcasted_iota(jnp.int32, (NQ, 256), 0) // 16
    slot_cols = []
    wsum_e = jnp.zeros((NQ, 128), jnp.float32)
    wsum_o = jnp.zeros((NQ, 128), jnp.float32)
    for lvl in range(L):
        gi = g_ref[lvl]
        ge = lax.bitcast_convert_type(gi << 16, jnp.float32)
        go = lax.bitcast_convert_type(gi & jnp.int32(-65536), jnp.float32)
        dre, dro = [], []
        for r in range(16):
            ke = ge[256 * r:256 * (r + 1), :]
            ko = go[256 * r:256 * (r + 1), :]
            dre.append(lax.dot_general(
                qp_ref[r], ke, (((1,), (0,)), ((), ())),
                preferred_element_type=jnp.float32).reshape(64, 1, 128))
            dro.append(lax.dot_general(
                qp_ref[r], ko, (((1,), (0,)), ((), ())),
                preferred_element_type=jnp.float32).reshape(64, 1, 128))
        d2e = jnp.concatenate(dre, axis=1).reshape(NQ, 128)
        d2o = jnp.concatenate(dro, axis=1).reshape(NQ, 128)
        t = lax.dot_general(d2e, awe_ref[lvl], (((1,), (1,)), ((), ())),
                            preferred_element_type=jnp.float32) + \
            lax.dot_general(d2o, awo_ref[lvl], (((1,), (1,)), ((), ())),
                            preferred_element_type=jnp.float32)
        cols = [jnp.sum(jnp.where(lane_iota == (q16 + p * 64), t, 0.0),
                        axis=1, keepdims=True) for p in range(4)]
        slot_cols.append(jnp.concatenate(cols, axis=1))
    ak = ak_ref[...]
    ki2 = lax.dot_general(ak, awx_ref[0], (((1,), (1,)), ((), ())),
                          preferred_element_type=jnp.float32)
    attn_add = lax.dot_general(q2, ki2, (((1,), (1,)), ((), ())),
                               preferred_element_type=jnp.float32)
    logits = jnp.concatenate(slot_cols + [attn_add], axis=1)
    m = jnp.max(logits, axis=1, keepdims=True)
    e = jnp.exp(logits - m)
    a = e / jnp.sum(e, axis=1, keepdims=True)
    for lvl in range(L):
        gi = g_ref[lvl]
        ge = lax.bitcast_convert_type(gi << 16, jnp.float32)
        go = lax.bitcast_convert_type(gi & jnp.int32(-65536), jnp.float32)
        for p in range(4):
            ac = a[:, lvl * 4 + p:lvl * 4 + p + 1]
            wsum_e = wsum_e + ac * ge[1024 * p:1024 * (p + 1), :]
            wsum_o = wsum_o + ac * go[1024 * p:1024 * (p + 1), :]
    a16 = jnp.sum(a[:, :16], axis=1, keepdims=True)
    v_main = lax.dot_general(wsum_e, vwe_ref[0], (((1,), (1,)), ((), ())),
                             preferred_element_type=jnp.float32) + \
        lax.dot_general(wsum_o, vwo_ref[0], (((1,), (1,)), ((), ())),
                        preferred_element_type=jnp.float32) + a16 * vb0_ref[0]
    v2 = lax.dot_general(ak, vw1_ref[0], (((1,), (1,)), ((), ())),
                         preferred_element_type=jnp.float32) + vb1_ref[0]
    v_add = lax.dot_general(a[:, 16:], v2, (((1,), (0,)), ((), ())),
                            preferred_element_type=jnp.float32)
    hm = hm_ref[...]
    hme = jnp.exp(hm - jnp.max(hm, axis=0, keepdims=True))
    hw = hme / jnp.sum(hme, axis=0, keepdims=True)
    hsel = lax.broadcasted_iota(jnp.int32, (9, 1), 0) == h + hoff
    hwh = jnp.sum(jnp.where(hsel, hw, 0.0), axis=0, keepdims=True)
    contrib = (v_main + v_add) * hwh

    @pl.when(h == 0)
    def _():
        base = acc_ref[...] + contrib
        if hoff == 0:
            base = base + q2 * hw[8:9, :]
        out_ref[...] = base

    @pl.when(h != 0)
    def _():
        out_ref[...] = out_ref[...] + contrib


def _compute_idx(q2, rp_t, w, b2d):
    return pl.pallas_call(
        _idx_body,
        out_shape=jax.ShapeDtypeStruct((32, 4, NQ), jnp.int32),
    )(q2, rp_t, w, b2d)


def _pack_body(x_ref, o_ref):
    x = x_ref[...]
    ei = lax.broadcasted_iota(jnp.int32, (256, 128), 0)
    ki = lax.broadcasted_iota(jnp.int32, (256, 128), 1)
    se = jnp.where(ei == 2 * ki, 1.0, 0.0).astype(jnp.float32)
    so = jnp.where(ei == 2 * ki + 1, 1.0, 0.0).astype(jnp.float32)
    ev = lax.dot_general(x, se, (((1,), (0,)), ((), ())),
                         preferred_element_type=jnp.float32)
    od = lax.dot_general(x, so, (((1,), (0,)), ((), ())),
                         preferred_element_type=jnp.float32)
    evi = lax.bitcast_convert_type(ev, jnp.int32)
    odi = lax.bitcast_convert_type(od, jnp.int32)
    evr = evi + 0x7FFF + ((evi >> 16) & 1)
    odr = odi + 0x7FFF + ((odi >> 16) & 1)
    o_ref[...] = lax.shift_right_logical(evr, 16) | (odr & jnp.int32(-65536))


def _pack_table(x):
    return pl.pallas_call(
        _pack_body,
        out_shape=jax.ShapeDtypeStruct((x.shape[0], 128), jnp.int32),
    )(x)


def _head_stage(hoff, nh, g, acc, q2, qp, aw_e, aw_o, attn_w, vw_e, vw_o,
                value_w, vb3, ak, hm_t):
    return pl.pallas_call(
        functools.partial(_head_body, hoff),
        grid=(nh,),
        in_specs=[
            pl.BlockSpec((4, 4096, 128), lambda h: (h, 0, 0)),
            pl.BlockSpec((NQ, 256), lambda h: (0, 0)),
            pl.BlockSpec((NQ, 256), lambda h: (0, 0)),
            pl.BlockSpec((16, 64, 256), lambda h: (0, 0, 0)),
            pl.BlockSpec((4, 256, 128), lambda h: (h + hoff, 0, 0)),
            pl.BlockSpec((4, 256, 128), lambda h: (h + hoff, 0, 0)),
            pl.BlockSpec((1, 256, 256), lambda h: (4 * (h + hoff) + 4, 0, 0)),
            pl.BlockSpec((1, 256, 128), lambda h: (2 * (h + hoff), 0, 0)),
            pl.BlockSpec((1, 256, 128), lambda h: (2 * (h + hoff), 0, 0)),
            pl.BlockSpec((1, 256, 256), lambda h: (2 * (h + hoff) + 1, 0, 0)),
            pl.BlockSpec((1, 1, 256), lambda h: (2 * (h + hoff), 0, 0)),
            pl.BlockSpec((1, 1, 256), lambda h: (2 * (h + hoff) + 1, 0, 0)),
            pl.BlockSpec((64, 256), lambda h: (0, 0)),
            pl.BlockSpec((9, 256), lambda h: (0, 0)),
        ],
        out_specs=pl.BlockSpec((NQ, 256), lambda h: (0, 0)),
        out_shape=jax.ShapeDtypeStruct((NQ, 256), jnp.float32),
    )(g, acc, q2, qp, aw_e, aw_o, attn_w, vw_e, vw_o, value_w, vb3, vb3,
      ak, hm_t)


def kernel(query, reference_points, input_flatten, input_spatial_shapes,
           add_keys, input_level_start_index, sampling_offsets_W,
           sampling_offsets_b, attn_W, value_W, value_b, head_mixer_W):
    q2 = query[0]
    rp_t = reference_points[0].transpose(1, 2, 0)
    b2d = sampling_offsets_b.reshape(256, 1)
    idx = _compute_idx(q2, rp_t, sampling_offsets_W, b2d)
    idxf = idx.reshape(NROWS)
    half = NROWS // 2
    gather = _make_sc_gather(half)
    table = _pack_table(input_flatten[0])
    g1 = gather(table, idxf[:half].reshape(32, 32, 64))
    g2 = gather(table, idxf[half:].reshape(32, 32, 64))
    qp = q2.reshape(64, 16, 256).transpose(1, 0, 2)
    aw_e, aw_o = attn_W[:, :, 0::2], attn_W[:, :, 1::2]
    vw_e, vw_o = value_W[:, :, 0::2], value_W[:, :, 1::2]
    vb3 = value_b.reshape(16, 1, 256)
    hm_t = head_mixer_W.T
    ak = add_keys[0]
    zero = jnp.zeros((NQ, 256), jnp.float32)
    o1 = _head_stage(0, 4, g1.reshape(16, 4096, 128), zero, q2, qp,
                     aw_e, aw_o, attn_W, vw_e, vw_o, value_W, vb3, ak, hm_t)
    out = _head_stage(4, 4, g2.reshape(16, 4096, 128), o1, q2, qp,
                      aw_e, aw_o, attn_W, vw_e, vw_o, value_W, vb3, ak, hm_t)
    return out[None]


# speedup vs baseline: 3.5655x; 2.6814x over previous
import functools

import jax
import jax.numpy as jnp
from jax import lax
from jax.experimental import pallas as pl
from jax.experimental.pallas import tpu as pltpu
from jax.experimental.pallas import tpu_sc as plsc

H, L, P, NQ, D = 8, 4, 4, 1024, 256
SLEV = (128.0, 64.0, 32.0, 16.0)
START = (0, 16384, 20480, 21504)
NROWS = 32 * 4096
SC_CH = 128
SC_NCH = 32


def _idx_body(q2_ref, rp_ref, w_ref, b_ref, out_ref):
    ot = lax.dot_general(w_ref[...], q2_ref[...], (((1,), (1,)), ((), ())),
                         preferred_element_type=jnp.float32) + b_ref[...]
    ot3 = ot.reshape(128, 2, NQ)
    for g in range(32):
        lvl = g % 4
        s = SLEV[lvl]
        xg = ot3[4 * g:4 * g + 4, 0, :]
        yg = ot3[4 * g:4 * g + 4, 1, :]
        lx = jnp.clip(rp_ref[lvl, 0:1, :] + xg * (1.0 / s), 0.0, 0.999)
        ly = jnp.clip(rp_ref[lvl, 1:2, :] + yg * (1.0 / s), 0.0, 0.999)
        ix = (lx * s).astype(jnp.int32)
        iy = (ly * s).astype(jnp.int32)
        out_ref[g] = ix + iy * int(s) + START[lvl]


@functools.lru_cache(maxsize=None)
def _make_sc_gather(nrows):
    ch = 64
    nbuf = 4
    nch = nrows // (32 * ch)
    rpw = nch * ch
    nit = nch // nbuf

    def body(table_hbm, idx_hbm, out_hbm, idx_v, b0, b1, b2, b3,
             si0, si1, si2, si3, so0, so1, so2, so3):
        bufs = (b0, b1, b2, b3)
        sin = (si0, si1, si2, si3)
        sout = (so0, so1, so2, so3)
        w = lax.axis_index("s") * 2 + lax.axis_index("c")
        pltpu.sync_copy(idx_hbm.at[w], idx_v)
        base = w * rpw
        for b in range(nbuf):
            pltpu.async_copy(table_hbm.at[idx_v.at[b]], bufs[b], sin[b])

        def quad(t, carry):
            cb = nbuf * t
            for b in range(nbuf):
                pltpu.make_async_copy(
                    table_hbm.at[idx_v.at[cb + b]], bufs[b], sin[b]).wait()
                pltpu.async_copy(
                    bufs[b], out_hbm.at[pl.ds(base + (cb + b) * ch, ch)],
                    sout[b])

            @pl.when(t < nit - 1)
            def _():
                for b in range(nbuf):
                    pltpu.make_async_copy(
                        bufs[b], out_hbm.at[pl.ds(base + (cb + b) * ch, ch)],
                        sout[b]).wait()
                    pltpu.async_copy(
                        table_hbm.at[idx_v.at[cb + b + nbuf]], bufs[b], sin[b])

            return carry

        lax.fori_loop(0, nit, quad, 0)
        for b in range(nbuf):
            c = nch - nbuf + b
            pltpu.make_async_copy(
                bufs[b], out_hbm.at[pl.ds(base + c * ch, ch)], sout[b]).wait()

    return functools.partial(
        pl.kernel,
        mesh=plsc.VectorSubcoreMesh(core_axis_name="c", subcore_axis_name="s"),
        out_type=jax.ShapeDtypeStruct((nrows, D // 2), jnp.int32),
        scratch_types=[pltpu.VMEM((nch, ch), jnp.int32)]
        + [pltpu.VMEM((ch, D // 2), jnp.int32)] * nbuf
        + [pltpu.SemaphoreType.DMA] * (2 * nbuf),
    )(body)


def _head_body(hoff, g_ref, acc_ref, q2_ref, qp_ref, aw_ref, awx_ref,
               vw0_ref, vw1_ref, vb0_ref, vb1_ref, ak_ref, hm_ref, out_ref):
    h = pl.program_id(0)
    q2 = q2_ref[...]
    ei = lax.broadcasted_iota(jnp.int32, (256, 128), 0)
    ki = lax.broadcasted_iota(jnp.int32, (256, 128), 1)
    se = jnp.where(ei == 2 * ki, 1.0, 0.0).astype(jnp.float32)
    so = jnp.where(ei == 2 * ki + 1, 1.0, 0.0).astype(jnp.float32)
    lane_iota = lax.broadcasted_iota(jnp.int32, (NQ, 256), 1)
    q16 = lax.broadcasted_iota(jnp.int32, (NQ, 256), 0) // 16
    slot_cols = []
    wsum_e = jnp.zeros((NQ, 128), jnp.float32)
    wsum_o = jnp.zeros((NQ, 128), jnp.float32)
    for lvl in range(L):
        gi = g_ref[lvl]
        ge = lax.bitcast_convert_type(gi << 16, jnp.float32)
        go = lax.bitcast_convert_type(gi & jnp.int32(-65536), jnp.float32)
        dre, dro = [], []
        for r in range(16):
            ke = ge[256 * r:256 * (r + 1), :]
            ko = go[256 * r:256 * (r + 1), :]
            dre.append(lax.dot_general(
                qp_ref[r], ke, (((1,), (0,)), ((), ())),
                preferred_element_type=jnp.float32).reshape(64, 1, 128))
            dro.append(lax.dot_general(
                qp_ref[r], ko, (((1,), (0,)), ((), ())),
                preferred_element_type=jnp.float32).reshape(64, 1, 128))
        d2e = jnp.concatenate(dre, axis=1).reshape(NQ, 128)
        d2o = jnp.concatenate(dro, axis=1).reshape(NQ, 128)
        sim_e = lax.dot_general(aw_ref[lvl], se, (((1,), (0,)), ((), ())),
                                preferred_element_type=jnp.float32)
        sim_o = lax.dot_general(aw_ref[lvl], so, (((1,), (0,)), ((), ())),
                                preferred_element_type=jnp.float32)
        t = lax.dot_general(d2e, sim_e, (((1,), (1,)), ((), ())),
                            preferred_element_type=jnp.float32) + \
            lax.dot_general(d2o, sim_o, (((1,), (1,)), ((), ())),
                            preferred_element_type=jnp.float32)
        cols = [jnp.sum(jnp.where(lane_iota == (q16 + p * 64), t, 0.0),
                        axis=1, keepdims=True) for p in range(4)]
        slot_cols.append(jnp.concatenate(cols, axis=1))
    ak = ak_ref[...]
    ki2 = lax.dot_general(ak, awx_ref[0], (((1,), (1,)), ((), ())),
                          preferred_element_type=jnp.float32)
    attn_add = lax.dot_general(q2, ki2, (((1,), (1,)), ((), ())),
                               preferred_element_type=jnp.float32)
    logits = jnp.concatenate(slot_cols + [attn_add], axis=1)
    m = jnp.max(logits, axis=1, keepdims=True)
    e = jnp.exp(logits - m)
    a = e / jnp.sum(e, axis=1, keepdims=True)
    for lvl in range(L):
        gi = g_ref[lvl]
        ge = lax.bitcast_convert_type(gi << 16, jnp.float32)
        go = lax.bitcast_convert_type(gi & jnp.int32(-65536), jnp.float32)
        for p in range(4):
            ac = a[:, lvl * 4 + p:lvl * 4 + p + 1]
            wsum_e = wsum_e + ac * ge[1024 * p:1024 * (p + 1), :]
            wsum_o = wsum_o + ac * go[1024 * p:1024 * (p + 1), :]
    a16 = jnp.sum(a[:, :16], axis=1, keepdims=True)
    vw_e = lax.dot_general(vw0_ref[0], se, (((1,), (0,)), ((), ())),
                           preferred_element_type=jnp.float32)
    vw_o = lax.dot_general(vw0_ref[0], so, (((1,), (0,)), ((), ())),
                           preferred_element_type=jnp.float32)
    v_main = lax.dot_general(wsum_e, vw_e, (((1,), (1,)), ((), ())),
                             preferred_element_type=jnp.float32) + \
        lax.dot_general(wsum_o, vw_o, (((1,), (1,)), ((), ())),
                        preferred_element_type=jnp.float32) + a16 * vb0_ref[0]
    v2 = lax.dot_general(ak, vw1_ref[0], (((1,), (1,)), ((), ())),
                         preferred_element_type=jnp.float32) + vb1_ref[0]
    v_add = lax.dot_general(a[:, 16:], v2, (((1,), (0,)), ((), ())),
                            preferred_element_type=jnp.float32)
    hm = hm_ref[...]
    hme = jnp.exp(hm - jnp.max(hm, axis=0, keepdims=True))
    hw = hme / jnp.sum(hme, axis=0, keepdims=True)
    hsel = lax.broadcasted_iota(jnp.int32, (9, 1), 0) == h + hoff
    hwh = jnp.sum(jnp.where(hsel, hw, 0.0), axis=0, keepdims=True)
    contrib = (v_main + v_add) * hwh

    @pl.when(h == 0)
    def _():
        base = acc_ref[...] + contrib
        if hoff == 0:
            base = base + q2 * hw[8:9, :]
        out_ref[...] = base

    @pl.when(h != 0)
    def _():
        out_ref[...] = out_ref[...] + contrib


def _compute_idx(q2, rp_t, w, b2d):
    return pl.pallas_call(
        _idx_body,
        out_shape=jax.ShapeDtypeStruct((32, 4, NQ), jnp.int32),
    )(q2, rp_t, w, b2d)


def _pack_body(x_ref, o_ref):
    x = x_ref[...]
    ei = lax.broadcasted_iota(jnp.int32, (256, 128), 0)
    ki = lax.broadcasted_iota(jnp.int32, (256, 128), 1)
    se = jnp.where(ei == 2 * ki, 1.0, 0.0).astype(jnp.float32)
    so = jnp.where(ei == 2 * ki + 1, 1.0, 0.0).astype(jnp.float32)
    ev = lax.dot_general(x, se, (((1,), (0,)), ((), ())),
                         preferred_element_type=jnp.float32)
    od = lax.dot_general(x, so, (((1,), (0,)), ((), ())),
                         preferred_element_type=jnp.float32)
    evi = lax.bitcast_convert_type(ev, jnp.int32)
    odi = lax.bitcast_convert_type(od, jnp.int32)
    evr = evi + 0x7FFF + ((evi >> 16) & 1)
    odr = odi + 0x7FFF + ((odi >> 16) & 1)
    o_ref[...] = lax.shift_right_logical(evr, 16) | (odr & jnp.int32(-65536))


def _pack_table(x):
    return pl.pallas_call(
        _pack_body,
        out_shape=jax.ShapeDtypeStruct((x.shape[0], 128), jnp.int32),
    )(x)


def _head_stage(hoff, nh, g, acc, q2, qp, attn_w, value_w, vb3, ak, hm_t):
    return pl.pallas_call(
        functools.partial(_head_body, hoff),
        grid=(nh,),
        in_specs=[
            pl.BlockSpec((4, 4096, 128), lambda h: (h, 0, 0)),
            pl.BlockSpec((NQ, 256), lambda h: (0, 0)),
            pl.BlockSpec((NQ, 256), lambda h: (0, 0)),
            pl.BlockSpec((16, 64, 256), lambda h: (0, 0, 0)),
            pl.BlockSpec((4, 256, 256), lambda h: (h + hoff, 0, 0)),
            pl.BlockSpec((1, 256, 256), lambda h: (4 * (h + hoff) + 4, 0, 0)),
            pl.BlockSpec((1, 256, 256), lambda h: (2 * (h + hoff), 0, 0)),
            pl.BlockSpec((1, 256, 256), lambda h: (2 * (h + hoff) + 1, 0, 0)),
            pl.BlockSpec((1, 1, 256), lambda h: (2 * (h + hoff), 0, 0)),
            pl.BlockSpec((1, 1, 256), lambda h: (2 * (h + hoff) + 1, 0, 0)),
            pl.BlockSpec((64, 256), lambda h: (0, 0)),
            pl.BlockSpec((9, 256), lambda h: (0, 0)),
        ],
        out_specs=pl.BlockSpec((NQ, 256), lambda h: (0, 0)),
        out_shape=jax.ShapeDtypeStruct((NQ, 256), jnp.float32),
    )(g, acc, q2, qp, attn_w, attn_w, value_w, value_w, vb3, vb3, ak, hm_t)


def kernel(query, reference_points, input_flatten, input_spatial_shapes,
           add_keys, input_level_start_index, sampling_offsets_W,
           sampling_offsets_b, attn_W, value_W, value_b, head_mixer_W):
    q2 = query[0]
    rp_t = reference_points[0].transpose(1, 2, 0)
    b2d = sampling_offsets_b.reshape(256, 1)
    idx = _compute_idx(q2, rp_t, sampling_offsets_W, b2d)
    idxf = idx.reshape(NROWS)
    half = NROWS // 2
    gather = _make_sc_gather(half)
    table = _pack_table(input_flatten[0])
    g1 = gather(table, idxf[:half].reshape(32, 32, 64))
    g2 = gather(table, idxf[half:].reshape(32, 32, 64))
    qp = q2.reshape(64, 16, 256).transpose(1, 0, 2)
    vb3 = value_b.reshape(16, 1, 256)
    hm_t = head_mixer_W.T
    ak = add_keys[0]
    zero = jnp.zeros((NQ, 256), jnp.float32)
    o1 = _head_stage(0, 4, g1.reshape(16, 4096, 128), zero, q2, qp,
                     attn_W, value_W, vb3, ak, hm_t)
    out = _head_stage(4, 4, g2.reshape(16, 4096, 128), o1, q2, qp,
                      attn_W, value_W, vb3, ak, hm_t)
    return out[None]


# speedup vs baseline: 3.7134x; 1.0415x over previous
import functools

import jax
import jax.numpy as jnp
from jax import lax
from jax.experimental import pallas as pl
from jax.experimental.pallas import tpu as pltpu
from jax.experimental.pallas import tpu_sc as plsc

H, L, P, NQ, D = 8, 4, 4, 1024, 256
SLEV = (128.0, 64.0, 32.0, 16.0)
START = (0, 16384, 20480, 21504)
NROWS = 32 * 4096
SC_CH = 128
SC_NCH = 32


def _idx_body(q2_ref, rp_ref, w_ref, b_ref, out_ref):
    ot = lax.dot_general(w_ref[...], q2_ref[...], (((1,), (1,)), ((), ())),
                         preferred_element_type=jnp.float32) + b_ref[...]
    ot3 = ot.reshape(128, 2, NQ)
    for g in range(32):
        lvl = g % 4
        s = SLEV[lvl]
        xg = ot3[4 * g:4 * g + 4, 0, :]
        yg = ot3[4 * g:4 * g + 4, 1, :]
        lx = jnp.clip(rp_ref[lvl, 0:1, :] + xg * (1.0 / s), 0.0, 0.999)
        ly = jnp.clip(rp_ref[lvl, 1:2, :] + yg * (1.0 / s), 0.0, 0.999)
        ix = (lx * s).astype(jnp.int32)
        iy = (ly * s).astype(jnp.int32)
        out_ref[g] = ix + iy * int(s) + START[lvl]


@functools.lru_cache(maxsize=None)
def _make_sc_gather(nrows):
    nch = nrows // (32 * SC_CH)
    rpw = nch * SC_CH

    def body(table_hbm, idx_hbm, out_hbm, idx_v, buf0, buf1,
             si0, si1, so0, so1):
        w = lax.axis_index("s") * 2 + lax.axis_index("c")
        pltpu.sync_copy(idx_hbm.at[w], idx_v)
        base = w * rpw
        pltpu.async_copy(table_hbm.at[idx_v.at[0]], buf0, si0)
        pltpu.async_copy(table_hbm.at[idx_v.at[1]], buf1, si1)

        def pair(t, carry):
            c0 = 2 * t
            pltpu.make_async_copy(table_hbm.at[idx_v.at[c0]], buf0, si0).wait()
            pltpu.async_copy(
                buf0, out_hbm.at[pl.ds(base + c0 * SC_CH, SC_CH)], so0)
            pltpu.make_async_copy(
                table_hbm.at[idx_v.at[c0 + 1]], buf1, si1).wait()
            pltpu.async_copy(
                buf1, out_hbm.at[pl.ds(base + (c0 + 1) * SC_CH, SC_CH)], so1)

            @pl.when(t < nch // 2 - 1)
            def _():
                pltpu.make_async_copy(
                    buf0, out_hbm.at[pl.ds(base + c0 * SC_CH, SC_CH)],
                    so0).wait()
                pltpu.async_copy(table_hbm.at[idx_v.at[c0 + 2]], buf0, si0)
                pltpu.make_async_copy(
                    buf1, out_hbm.at[pl.ds(base + (c0 + 1) * SC_CH, SC_CH)],
                    so1).wait()
                pltpu.async_copy(table_hbm.at[idx_v.at[c0 + 3]], buf1, si1)

            return carry

        lax.fori_loop(0, nch // 2, pair, 0)
        last = nch - 2
        pltpu.make_async_copy(
            buf0, out_hbm.at[pl.ds(base + last * SC_CH, SC_CH)], so0).wait()
        pltpu.make_async_copy(
            buf1, out_hbm.at[pl.ds(base + (last + 1) * SC_CH, SC_CH)],
            so1).wait()

    return functools.partial(
        pl.kernel,
        mesh=plsc.VectorSubcoreMesh(core_axis_name="c", subcore_axis_name="s"),
        out_type=jax.ShapeDtypeStruct((nrows, D), jnp.float32),
        scratch_types=[
            pltpu.VMEM((nch, SC_CH), jnp.int32),
            pltpu.VMEM((SC_CH, D), jnp.float32),
            pltpu.VMEM((SC_CH, D), jnp.float32),
            pltpu.SemaphoreType.DMA,
            pltpu.SemaphoreType.DMA,
            pltpu.SemaphoreType.DMA,
            pltpu.SemaphoreType.DMA,
        ],
    )(body)


def _head_body(hoff, g_ref, acc_ref, q2_ref, qp_ref, aw_ref, awx_ref,
               vw0_ref, vw1_ref, vb0_ref, vb1_ref, ak_ref, hm_ref, out_ref):
    h = pl.program_id(0)
    q2 = q2_ref[...]
    lane_iota = lax.broadcasted_iota(jnp.int32, (NQ, 256), 1)
    q16 = lax.broadcasted_iota(jnp.int32, (NQ, 256), 0) // 16
    slot_cols = []
    for lvl in range(L):
        simil = aw_ref[lvl]
        drs = []
        for r in range(16):
            kr = g_ref[lvl, 256 * r:256 * (r + 1), :]
            dr = lax.dot_general(qp_ref[r], kr, (((1,), (0,)), ((), ())),
                                 preferred_element_type=jnp.float32)
            drs.append(dr.reshape(64, 1, 256))
        d2 = jnp.concatenate(drs, axis=1).reshape(NQ, 256)
        t = lax.dot_general(d2, simil, (((1,), (1,)), ((), ())),
                            preferred_element_type=jnp.float32)
        cols = [jnp.sum(jnp.where(lane_iota == (q16 + p * 64), t, 0.0),
                        axis=1, keepdims=True) for p in range(4)]
        slot_cols.append(jnp.concatenate(cols, axis=1))
    ak = ak_ref[...]
    ki2 = lax.dot_general(ak, awx_ref[0], (((1,), (1,)), ((), ())),
                          preferred_element_type=jnp.float32)
    attn_add = lax.dot_general(q2, ki2, (((1,), (1,)), ((), ())),
                               preferred_element_type=jnp.float32)
    logits = jnp.concatenate(slot_cols + [attn_add], axis=1)
    m = jnp.max(logits, axis=1, keepdims=True)
    e = jnp.exp(logits - m)
    a = e / jnp.sum(e, axis=1, keepdims=True)
    wsum = jnp.zeros((NQ, 256), jnp.float32)
    for lvl in range(L):
        for p in range(4):
            wsum = wsum + a[:, lvl * 4 + p:lvl * 4 + p + 1] * \
                g_ref[lvl, 1024 * p:1024 * (p + 1), :]
    a16 = jnp.sum(a[:, :16], axis=1, keepdims=True)
    v_main = lax.dot_general(wsum, vw0_ref[0], (((1,), (1,)), ((), ())),
                             preferred_element_type=jnp.float32) + a16 * vb0_ref[0]
    v2 = lax.dot_general(ak, vw1_ref[0], (((1,), (1,)), ((), ())),
                         preferred_element_type=jnp.float32) + vb1_ref[0]
    v_add = lax.dot_general(a[:, 16:], v2, (((1,), (0,)), ((), ())),
                            preferred_element_type=jnp.float32)
    hm = hm_ref[...]
    hme = jnp.exp(hm - jnp.max(hm, axis=0, keepdims=True))
    hw = hme / jnp.sum(hme, axis=0, keepdims=True)
    hsel = lax.broadcasted_iota(jnp.int32, (9, 1), 0) == h + hoff
    hwh = jnp.sum(jnp.where(hsel, hw, 0.0), axis=0, keepdims=True)
    contrib = (v_main + v_add) * hwh

    @pl.when(h == 0)
    def _():
        base = acc_ref[...] + contrib
        if hoff == 0:
            base = base + q2 * hw[8:9, :]
        out_ref[...] = base

    @pl.when(h != 0)
    def _():
        out_ref[...] = out_ref[...] + contrib


def _compute_idx(q2, rp_t, w, b2d):
    return pl.pallas_call(
        _idx_body,
        out_shape=jax.ShapeDtypeStruct((32, 4, NQ), jnp.int32),
    )(q2, rp_t, w, b2d)


def _head_stage(hoff, nh, g, acc, q2, qp, attn_w, value_w, vb3, ak, hm_t):
    return pl.pallas_call(
        functools.partial(_head_body, hoff),
        grid=(nh,),
        in_specs=[
            pl.BlockSpec((4, 4096, 256), lambda h: (h, 0, 0)),
            pl.BlockSpec((NQ, 256), lambda h: (0, 0)),
            pl.BlockSpec((NQ, 256), lambda h: (0, 0)),
            pl.BlockSpec((16, 64, 256), lambda h: (0, 0, 0)),
            pl.BlockSpec((4, 256, 256), lambda h: (h + hoff, 0, 0)),
            pl.BlockSpec((1, 256, 256), lambda h: (4 * (h + hoff) + 4, 0, 0)),
            pl.BlockSpec((1, 256, 256), lambda h: (2 * (h + hoff), 0, 0)),
            pl.BlockSpec((1, 256, 256), lambda h: (2 * (h + hoff) + 1, 0, 0)),
            pl.BlockSpec((1, 1, 256), lambda h: (2 * (h + hoff), 0, 0)),
            pl.BlockSpec((1, 1, 256), lambda h: (2 * (h + hoff) + 1, 0, 0)),
            pl.BlockSpec((64, 256), lambda h: (0, 0)),
            pl.BlockSpec((9, 256), lambda h: (0, 0)),
        ],
        out_specs=pl.BlockSpec((NQ, 256), lambda h: (0, 0)),
        out_shape=jax.ShapeDtypeStruct((NQ, 256), jnp.float32),
    )(g, acc, q2, qp, attn_w, attn_w, value_w, value_w, vb3, vb3, ak, hm_t)


def kernel(query, reference_points, input_flatten, input_spatial_shapes,
           add_keys, input_level_start_index, sampling_offsets_W,
           sampling_offsets_b, attn_W, value_W, value_b, head_mixer_W):
    q2 = query[0]
    rp_t = reference_points[0].transpose(1, 2, 0)
    b2d = sampling_offsets_b.reshape(256, 1)
    idx = _compute_idx(q2, rp_t, sampling_offsets_W, b2d)
    idxf = idx.reshape(NROWS)
    half = NROWS // 2
    gather = _make_sc_gather(half)
    table = input_flatten[0]
    g1 = gather(table, idxf[:half].reshape(32, 16, SC_CH))
    g2 = gather(table, idxf[half:].reshape(32, 16, SC_CH))
    qp = q2.reshape(64, 16, 256).transpose(1, 0, 2)
    vb3 = value_b.reshape(16, 1, 256)
    hm_t = head_mixer_W.T
    ak = add_keys[0]
    zero = jnp.zeros((NQ, 256), jnp.float32)
    o1 = _head_stage(0, 4, g1.reshape(16, 4096, 256), zero, q2, qp,
                     attn_W, value_W, vb3, ak, hm_t)
    out = _head_stage(4, 4, g2.reshape(16, 4096, 256), o1, q2, qp,
                      attn_W, value_W, vb3, ak, hm_t)
    return out[None]
